# baseline (speedup 1.0000x reference)
import functools

import numpy as np
import jax
import jax.numpy as jnp
from jax.experimental import pallas as pl
from jax.experimental.pallas import tpu as pltpu
from jax.experimental.pallas import tpu_sc as plsc

F32 = jnp.float32
BF16 = jnp.bfloat16
_HI = jax.lax.Precision.HIGHEST

_B = 32
_T = 64
_TQ = 16
_NB = 512
_CD = 256


def _sep_perms():
    pos0, rot0, vel0, foot0, nch = 4, 67, 193, 259, 263
    lower = np.array([0, 1, 2, 4, 5, 7, 8, 10, 11])
    lower_ex = lower[1:] - 1
    upper = np.array([3, 6, 9, 12, 13, 14, 15, 16, 17, 18, 19, 20, 21])
    upper_ex = upper - 1
    up = np.concatenate([
        (pos0 + upper_ex[:, None] * 3 + np.arange(3)).ravel(),
        (rot0 + upper_ex[:, None] * 6 + np.arange(6)).ravel(),
        (vel0 + upper[:, None] * 3 + np.arange(3)).ravel(),
    ])
    low = np.concatenate([
        np.arange(4),
        (pos0 + lower_ex[:, None] * 3 + np.arange(3)).ravel(),
        (rot0 + lower_ex[:, None] * 6 + np.arange(6)).ravel(),
        (vel0 + lower[:, None] * 3 + np.arange(3)).ravel(),
        np.arange(foot0, nch),
    ])
    return up, low


_PERM_UP, _PERM_LOW = _sep_perms()


def _taps(w):
    o, i, k = w.shape
    return jnp.transpose(w, (2, 1, 0)).reshape(k * i, o).astype(BF16)


def _bias(b):
    return b.reshape(1, -1)


def _dotb(x, w):
    return jax.lax.dot(x, w, preferred_element_type=F32)


def _conv(x, wt, b, dil, padn):
    tq, bq, c = x.shape
    k = wt.shape[0] // c
    co = wt.shape[1]
    xb = x.astype(BF16)
    xp = jnp.pad(xb, ((padn, padn), (0, 0), (0, 0))) if padn else xb
    parts = [
        jax.lax.slice(xp, (i * dil, 0, 0),
                      (i * dil + tq, bq, c)).reshape(tq * bq, c)
        for i in range(k)
    ]
    op = parts[0] if k == 1 else jnp.concatenate(parts, axis=1)
    y = _dotb(op, wt)
    return (y + b).reshape(tq, bq, co)


def _down(x, wt, b):
    tq, bq, c = x.shape
    to = tq // 2
    xb = jnp.pad(x.astype(BF16), ((1, 1), (0, 0), (0, 0)))
    r = xb.reshape(to + 1, 2, bq, c)
    parts = [p.reshape(to * bq, c)
             for p in (r[0:to, 0], r[0:to, 1], r[1:, 0], r[1:, 1])]
    y = _dotb(jnp.concatenate(parts, axis=1), wt)
    return (y + b).reshape(to, bq, wt.shape[1])


def _enc_fwd(x, ws):
    it = iter(ws)
    h = _conv(x, next(it), next(it), 1, 1)
    h = jnp.maximum(h, 0.0)
    for _ in range(2):
        h = _down(h, next(it), next(it))
        for j in range(3):
            w1, b1, w2, b2 = next(it), next(it), next(it), next(it)
            g = jnp.maximum(h, 0.0)
            g = _conv(g, w1, b1, 3 ** j, 3 ** j)
            g = jnp.maximum(g, 0.0)
            g = _conv(g, w2, b2, 1, 0)
            h = h + g
    h = _conv(h, next(it), next(it), 1, 1)
    return h


def _argmin_codes(xf, cb, cbt):
    r, n = xf.shape[0], cb.shape[0]
    x2 = jnp.sum(xf * xf, axis=1, keepdims=True)
    c2 = jnp.sum(cb * cb, axis=1)
    xy = _dotb(xf.astype(BF16), cbt)
    dist = (x2 - 2.0 * xy) + c2[None, :]
    m = jnp.min(dist, axis=1, keepdims=True)
    iota = jax.lax.broadcasted_iota(jnp.int32, (r, n), 1)
    idx = jnp.min(jnp.where(dist == m, iota, n), axis=1)
    return idx


def _enc_body(nw, *refs):
    x = refs[0][...]
    ws = [rr[...] for rr in refs[1:1 + nw]]
    cb = refs[1 + nw][...]
    cbt = refs[2 + nw][...]
    xf_ref, idx_ref = refs[3 + nw:5 + nw]
    h = _enc_fwd(x, ws)
    xf = h.reshape(_TQ * _B, _CD)
    xf_ref[...] = xf
    idx_ref[...] = _argmin_codes(xf, cb, cbt)


def _sc_gather_call(cb_up, cb_low, idx_up, idx_low):
    info = plsc.get_sparse_core_info()
    nw = info.num_cores * info.num_subcores
    r = _TQ * _B
    bpw = r // nw
    mesh = plsc.VectorSubcoreMesh(core_axis_name="c", subcore_axis_name="s")

    @functools.partial(
        pl.kernel, mesh=mesh,
        out_type=(jax.ShapeDtypeStruct((r, _CD), F32),
                  jax.ShapeDtypeStruct((r, _CD), F32)),
        scratch_types=[pltpu.VMEM((bpw,), jnp.int32),
                       pltpu.VMEM((bpw, _CD), F32),
                       pltpu.SemaphoreType.DMA],
    )
    def k(cbu, cbl, iu, il, ou, ol, idx_v, rows_v, sem):
        wid = jax.lax.axis_index("s") * info.num_cores + jax.lax.axis_index("c")
        base = wid * bpw
        pltpu.sync_copy(iu.at[pl.ds(base, bpw)], idx_v)
        pltpu.async_copy(cbu.at[idx_v], rows_v, sem).wait()
        pltpu.sync_copy(rows_v, ou.at[pl.ds(base, bpw)])
        pltpu.sync_copy(il.at[pl.ds(base, bpw)], idx_v)
        pltpu.async_copy(cbl.at[idx_v], rows_v, sem).wait()
        pltpu.sync_copy(rows_v, ol.at[pl.ds(base, bpw)])

    return k(cb_up, cb_low, idx_up, idx_low)


def _dec_fwd(x, ws):
    it = iter(ws)
    h = _conv(x, next(it), next(it), 1, 1)
    h = jnp.maximum(h, 0.0)
    for _ in range(2):
        for j in range(3):
            w1, b1, w2, b2 = next(it), next(it), next(it), next(it)
            d = 3 ** (2 - j)
            g = jnp.maximum(h, 0.0)
            g = _conv(g, w1, b1, d, d)
            g = jnp.maximum(g, 0.0)
            g = _conv(g, w2, b2, 1, 0)
            h = h + g
        wu, bu = next(it), next(it)
        tq, bq, c = h.shape
        hr = jnp.stack([h, h], axis=1).reshape(2 * tq, bq, c)
        h = _conv(hr, wu, bu, 1, 1)
    h = _conv(h, next(it), next(it), 1, 1)
    h = jnp.maximum(h, 0.0)
    h = _conv(h, next(it), next(it), 1, 1)
    return h


def _dec_body(nw, *refs):
    x = refs[0][...]
    xfu = refs[1][...]
    xfl = refs[2][...]
    idxl = refs[3][...]
    ws = [rr[...] for rr in refs[4:4 + nw]]
    out_ref, loss_ref, ppl_ref = refs[4 + nw:7 + nw]
    out_ref[...] = _dec_fwd(x, ws)
    r = _TQ * _B
    xq = x.reshape(r, 2 * _CD)
    cu = jnp.mean((xfu - xq[:, :_CD]) ** 2)
    cl = jnp.mean((xfl - xq[:, _CD:]) ** 2)
    loss_ref[...] = (cu + cl).reshape(1, 1)
    iota = jax.lax.broadcasted_iota(jnp.int32, (r, _NB), 1)
    counts = jnp.sum((iota == idxl[:, None]).astype(F32), axis=0)
    prob = counts / r
    ppl = jnp.exp(-jnp.sum(prob * jnp.log(prob + 1e-7)))
    ppl_ref[...] = ppl.reshape(1, 1)


def _enc_wlist(p, cpad):
    w_in = p['conv_in']['w']
    w_in = jnp.pad(w_in, ((0, 0), (0, cpad - w_in.shape[1]), (0, 0)))
    out = [_taps(w_in), _bias(p['conv_in']['b'])]
    for blk in p['downs']:
        out += [_taps(blk['down']['w']), _bias(blk['down']['b'])]
        for rb in blk['res']:
            out += [_taps(rb['c1']['w']), _bias(rb['c1']['b']),
                    _taps(rb['c2']['w']), _bias(rb['c2']['b'])]
    out += [_taps(p['conv_out']['w']), _bias(p['conv_out']['b'])]
    return out


def _dec_wlist(p):
    out = [_taps(p['conv_in']['w']), _bias(p['conv_in']['b'])]
    for blk in p['ups']:
        for rb in blk['res']:
            out += [_taps(rb['c1']['w']), _bias(rb['c1']['b']),
                    _taps(rb['c2']['w']), _bias(rb['c2']['b'])]
        out += [_taps(blk['up']['w']), _bias(blk['up']['b'])]
    out += [_taps(p['conv_mid']['w']), _bias(p['conv_mid']['b']),
            _taps(p['conv_out']['w']), _bias(p['conv_out']['b'])]
    return out


def _enc_call(x, wlist, cb):
    cbt = jnp.transpose(cb).astype(BF16)
    nw = len(wlist)
    out_shape = (jax.ShapeDtypeStruct((_TQ * _B, _CD), F32),
                 jax.ShapeDtypeStruct((_TQ * _B,), jnp.int32))
    return pl.pallas_call(
        functools.partial(_enc_body, nw), out_shape=out_shape,
    )(x, *wlist, cb, cbt)


def _dec_call(xq, xfu, xfl, idxl, wlist):
    nw = len(wlist)
    out_shape = (jax.ShapeDtypeStruct((_T, _B, 263), F32),
                 jax.ShapeDtypeStruct((1, 1), F32),
                 jax.ShapeDtypeStruct((1, 1), F32))
    return pl.pallas_call(
        functools.partial(_dec_body, nw), out_shape=out_shape,
    )(xq, xfu, xfl, idxl, *wlist)


def kernel(x, params):
    xt = jnp.transpose(x, (1, 0, 2))
    xtu = jnp.take(xt, _PERM_UP, axis=2)
    xtl = jnp.take(xt, _PERM_LOW, axis=2)
    xtu = jnp.pad(xtu, ((0, 0), (0, 0), (0, -xtu.shape[2] % 8)))
    xtl = jnp.pad(xtl, ((0, 0), (0, 0), (0, -xtl.shape[2] % 8)))
    wu = _enc_wlist(params['enc_up'], xtu.shape[2])
    wl = _enc_wlist(params['enc_low'], xtl.shape[2])
    xfu, idxu = _enc_call(xtu, wu, params['cb_up'])
    xfl, idxl = _enc_call(xtl, wl, params['cb_low'])
    xdu, xdl = _sc_gather_call(params['cb_up'], params['cb_low'], idxu, idxl)
    xq = jnp.concatenate([xdu, xdl], axis=-1).reshape(_TQ, _B, 2 * _CD)
    xout, loss, ppl = _dec_call(xq, xfu, xfl, idxl, _dec_wlist(params['dec']))
    return jnp.transpose(xout, (1, 0, 2)), loss[0, 0], ppl[0, 0]


# speedup vs baseline: 1.0009x; 1.0009x over previous
import functools

import numpy as np
import jax
import jax.numpy as jnp
from jax.experimental import pallas as pl
from jax.experimental.pallas import tpu as pltpu
from jax.experimental.pallas import tpu_sc as plsc

F32 = jnp.float32
BF16 = jnp.bfloat16

_B = 32
_T = 64
_TQ = 16
_NB = 512
_CD = 256


def _sep_perms():
    pos0, rot0, vel0, foot0, nch = 4, 67, 193, 259, 263
    lower = np.array([0, 1, 2, 4, 5, 7, 8, 10, 11])
    lower_ex = lower[1:] - 1
    upper = np.array([3, 6, 9, 12, 13, 14, 15, 16, 17, 18, 19, 20, 21])
    upper_ex = upper - 1
    up = np.concatenate([
        (pos0 + upper_ex[:, None] * 3 + np.arange(3)).ravel(),
        (rot0 + upper_ex[:, None] * 6 + np.arange(6)).ravel(),
        (vel0 + upper[:, None] * 3 + np.arange(3)).ravel(),
    ])
    low = np.concatenate([
        np.arange(4),
        (pos0 + lower_ex[:, None] * 3 + np.arange(3)).ravel(),
        (rot0 + lower_ex[:, None] * 6 + np.arange(6)).ravel(),
        (vel0 + lower[:, None] * 3 + np.arange(3)).ravel(),
        np.arange(foot0, nch),
    ])
    return up, low


_PERM_UP, _PERM_LOW = _sep_perms()


def _taps(w):
    o, i, k = w.shape
    return jnp.transpose(w, (2, 1, 0)).reshape(k * i, o).astype(BF16)


def _bias(b):
    return b.reshape(1, -1)


def _dotb(x, w):
    return jax.lax.dot(x, w, preferred_element_type=F32)


def _conv(x, wt, b, dil, padn):
    tq, bq, c = x.shape
    k = wt.shape[0] // c
    co = wt.shape[1]
    xb = x.astype(BF16)
    xp = jnp.pad(xb, ((padn, padn), (0, 0), (0, 0))) if padn else xb
    parts = [
        jax.lax.slice(xp, (i * dil, 0, 0),
                      (i * dil + tq, bq, c)).reshape(tq * bq, c)
        for i in range(k)
    ]
    op = parts[0] if k == 1 else jnp.concatenate(parts, axis=1)
    y = _dotb(op, wt)
    return (y + b).reshape(tq, bq, co)


def _down(x, wt, b):
    tq, bq, c = x.shape
    to = tq // 2
    xb = jnp.pad(x.astype(BF16), ((1, 1), (0, 0), (0, 0)))
    r = xb.reshape(to + 1, 2, bq, c)
    parts = [p.reshape(to * bq, c)
             for p in (r[0:to, 0], r[0:to, 1], r[1:, 0], r[1:, 1])]
    y = _dotb(jnp.concatenate(parts, axis=1), wt)
    return (y + b).reshape(to, bq, wt.shape[1])


def _enc_fwd(x, ws):
    it = iter(ws)
    h = _conv(x, next(it), next(it), 1, 1)
    h = jnp.maximum(h, 0.0)
    for _ in range(2):
        h = _down(h, next(it), next(it))
        for j in range(3):
            w1, b1, w2, b2 = next(it), next(it), next(it), next(it)
            g = jnp.maximum(h, 0.0)
            g = _conv(g, w1, b1, 3 ** j, 3 ** j)
            g = jnp.maximum(g, 0.0)
            g = _conv(g, w2, b2, 1, 0)
            h = h + g
    h = _conv(h, next(it), next(it), 1, 1)
    return h


def _argmin_codes(xf, cb, cbt):
    r, n = xf.shape[0], cb.shape[0]
    x2 = jnp.sum(xf * xf, axis=1, keepdims=True)
    c2 = jnp.sum(cb * cb, axis=1)
    xy = _dotb(xf.astype(BF16), cbt)
    dist = (x2 - 2.0 * xy) + c2[None, :]
    m = jnp.min(dist, axis=1, keepdims=True)
    iota = jax.lax.broadcasted_iota(jnp.int32, (r, n), 1)
    idx = jnp.min(jnp.where(dist == m, iota, n), axis=1)
    return idx


def _enc_body(nw, *refs):
    x = refs[0][...]
    ws = [rr[...] for rr in refs[1:1 + nw]]
    cb = refs[1 + nw][...]
    cbt = refs[2 + nw][...]
    xf_ref, idx_ref = refs[3 + nw:5 + nw]
    h = _enc_fwd(x, ws)
    xf = h.reshape(_TQ * _B, _CD)
    xf_ref[...] = xf
    idx_ref[...] = _argmin_codes(xf, cb, cbt)


def _sc_gather_call(cb_up, cb_low, idx_up, idx_low):
    info = plsc.get_sparse_core_info()
    nw = info.num_cores * info.num_subcores
    r = _TQ * _B
    bpw = r // nw
    mesh = plsc.VectorSubcoreMesh(core_axis_name="c", subcore_axis_name="s")

    @functools.partial(
        pl.kernel, mesh=mesh,
        out_type=(jax.ShapeDtypeStruct((r, _CD), F32),
                  jax.ShapeDtypeStruct((r, _CD), F32)),
        scratch_types=[pltpu.VMEM((bpw,), jnp.int32),
                       pltpu.VMEM((bpw, _CD), F32),
                       pltpu.SemaphoreType.DMA],
    )
    def k(cbu, cbl, iu, il, ou, ol, idx_v, rows_v, sem):
        wid = jax.lax.axis_index("s") * info.num_cores + jax.lax.axis_index("c")
        base = wid * bpw
        pltpu.sync_copy(iu.at[pl.ds(base, bpw)], idx_v)
        pltpu.async_copy(cbu.at[idx_v], rows_v, sem).wait()
        pltpu.sync_copy(rows_v, ou.at[pl.ds(base, bpw)])
        pltpu.sync_copy(il.at[pl.ds(base, bpw)], idx_v)
        pltpu.async_copy(cbl.at[idx_v], rows_v, sem).wait()
        pltpu.sync_copy(rows_v, ol.at[pl.ds(base, bpw)])

    return k(cb_up, cb_low, idx_up, idx_low)


def _dec_fwd(x, ws):
    it = iter(ws)
    h = _conv(x, next(it), next(it), 1, 1)
    h = jnp.maximum(h, 0.0)
    for _ in range(2):
        for j in range(3):
            w1, b1, w2, b2 = next(it), next(it), next(it), next(it)
            d = 3 ** (2 - j)
            g = jnp.maximum(h, 0.0)
            g = _conv(g, w1, b1, d, d)
            g = jnp.maximum(g, 0.0)
            g = _conv(g, w2, b2, 1, 0)
            h = h + g
        wu, bu = next(it), next(it)
        tq, bq, c = h.shape
        hr = jnp.stack([h, h], axis=1).reshape(2 * tq, bq, c)
        h = _conv(hr, wu, bu, 1, 1)
    h = _conv(h, next(it), next(it), 1, 1)
    h = jnp.maximum(h, 0.0)
    h = _conv(h, next(it), next(it), 1, 1)
    return h


def _dec_body(nw, *refs):
    x = refs[0][...]
    xfu = refs[1][...]
    xfl = refs[2][...]
    idxl = refs[3][...]
    ws = [rr[...] for rr in refs[4:4 + nw]]
    out_ref, loss_ref, ppl_ref = refs[4 + nw:7 + nw]
    out_ref[...] = _dec_fwd(x, ws)
    r = _TQ * _B
    xq = x.reshape(r, 2 * _CD)
    cu = jnp.mean((xfu - xq[:, :_CD]) ** 2)
    cl = jnp.mean((xfl - xq[:, _CD:]) ** 2)
    loss_ref[...] = (cu + cl).reshape(1, 1)
    iota = jax.lax.broadcasted_iota(jnp.int32, (r, _NB), 1)
    counts = jnp.sum((iota == idxl[:, None]).astype(F32), axis=0)
    prob = counts / r
    ppl = jnp.exp(-jnp.sum(prob * jnp.log(prob + 1e-7)))
    ppl_ref[...] = ppl.reshape(1, 1)


def _enc_wlist(p, cpad):
    w_in = p['conv_in']['w']
    w_in = jnp.pad(w_in, ((0, 0), (0, cpad - w_in.shape[1]), (0, 0)))
    out = [_taps(w_in), _bias(p['conv_in']['b'])]
    for blk in p['downs']:
        out += [_taps(blk['down']['w']), _bias(blk['down']['b'])]
        for rb in blk['res']:
            out += [_taps(rb['c1']['w']), _bias(rb['c1']['b']),
                    _taps(rb['c2']['w']), _bias(rb['c2']['b'])]
    out += [_taps(p['conv_out']['w']), _bias(p['conv_out']['b'])]
    return out


def _dec_wlist(p):
    out = [_taps(p['conv_in']['w']), _bias(p['conv_in']['b'])]
    for blk in p['ups']:
        for rb in blk['res']:
            out += [_taps(rb['c1']['w']), _bias(rb['c1']['b']),
                    _taps(rb['c2']['w']), _bias(rb['c2']['b'])]
        out += [_taps(blk['up']['w']), _bias(blk['up']['b'])]
    out += [_taps(p['conv_mid']['w']), _bias(p['conv_mid']['b']),
            _taps(p['conv_out']['w']), _bias(p['conv_out']['b'])]
    return out


def _enc_call(x, wlist, cb):
    cbt = jnp.transpose(cb).astype(BF16)
    nw = len(wlist)
    out_shape = (jax.ShapeDtypeStruct((_TQ * _B, _CD), F32),
                 jax.ShapeDtypeStruct((_TQ * _B,), jnp.int32))
    return pl.pallas_call(
        functools.partial(_enc_body, nw), out_shape=out_shape,
    )(x, *wlist, cb, cbt)


def _dec_call(xq, xfu, xfl, idxl, wlist):
    nw = len(wlist)
    out_shape = (jax.ShapeDtypeStruct((_T, _B, 263), F32),
                 jax.ShapeDtypeStruct((1, 1), F32),
                 jax.ShapeDtypeStruct((1, 1), F32))
    return pl.pallas_call(
        functools.partial(_dec_body, nw), out_shape=out_shape,
    )(xq, xfu, xfl, idxl, *wlist)


def kernel(x, params):
    xt = jnp.transpose(x, (1, 0, 2))
    xtu = jnp.take(xt, _PERM_UP, axis=2)
    xtl = jnp.take(xt, _PERM_LOW, axis=2)
    xtu = jnp.pad(xtu, ((0, 0), (0, 0), (0, -xtu.shape[2] % 8)))
    xtl = jnp.pad(xtl, ((0, 0), (0, 0), (0, -xtl.shape[2] % 8)))
    wu = _enc_wlist(params['enc_up'], xtu.shape[2])
    wl = _enc_wlist(params['enc_low'], xtl.shape[2])
    xfu, idxu = _enc_call(xtu, wu, params['cb_up'])
    xfl, idxl = _enc_call(xtl, wl, params['cb_low'])
    xdu, xdl = _sc_gather_call(params['cb_up'], params['cb_low'], idxu, idxl)
    xq = jnp.concatenate([xdu, xdl], axis=-1).reshape(_TQ, _B, 2 * _CD)
    xout, loss, ppl = _dec_call(xq, xfu, xfl, idxl, _dec_wlist(params['dec']))
    return jnp.transpose(xout, (1, 0, 2)), loss[0, 0], ppl[0, 0]
